# baseline (speedup 1.0000x reference)
import jax
import jax.numpy as jnp
from jax.experimental import pallas as pl
from jax.experimental.pallas import tpu as pltpu

N_ROWS = 64
C = 256
_HW = (56 * 56, 28 * 28, 14 * 14, 7 * 7, 4 * 4)


def _scatter_kernel(layer_ref, ch_ref, i0, i1, i2, i3, i4,
                    o0, o1, o2, o3, o4, f0, f1, f2, f3, f4, sem):
    del i0, i1, i2, i3, i4
    outs = (o0, o1, o2, o3, o4)
    fills = (f0, f1, f2, f3, f4)
    for f, hw in zip(fills, _HW):
        f[...] = jnp.full((hw,), -1000.0, jnp.float32)
    def start_one(li, dest):
        pltpu.make_async_copy(fills[li], dest, sem).start()
    def wait_one(li, dest):
        pltpu.make_async_copy(fills[li], dest, sem).wait()
    for i in range(N_ROWS):
        lid = layer_ref[i]
        c = ch_ref[i]
        for li in range(5):
            @pl.when(lid == li)
            def _(li=li, i=i, c=c):
                start_one(li, outs[li].at[i * C + c])
    for i in range(N_ROWS):
        lid = layer_ref[i]
        c = ch_ref[i]
        for li in range(5):
            @pl.when(lid == li)
            def _(li=li, i=i, c=c):
                wait_one(li, outs[li].at[i * C + c])


def kernel(act_0, act_1, act_2, act_3, act_pool, indices, x):
    del x
    acts = (act_0, act_1, act_2, act_3, act_pool)
    layer_ids = (indices // C).astype(jnp.int32)
    ch = (indices % C).astype(jnp.int32)
    flat = [a.reshape(N_ROWS * C, hw) for a, hw in zip(acts, _HW)]

    any_spec = pl.BlockSpec(memory_space=pl.ANY)
    smem_spec = pl.BlockSpec(memory_space=pltpu.SMEM)

    outs = pl.pallas_call(
        _scatter_kernel,
        in_specs=[smem_spec, smem_spec] + [any_spec] * 5,
        out_specs=[any_spec] * 5,
        out_shape=[jax.ShapeDtypeStruct(f.shape, f.dtype) for f in flat],
        input_output_aliases={2: 0, 3: 1, 4: 2, 5: 3, 6: 4},
        scratch_shapes=[pltpu.VMEM((hw,), jnp.float32) for hw in _HW]
        + [pltpu.SemaphoreType.DMA],
    )(layer_ids, ch, *flat)

    return tuple(o.reshape(a.shape) for o, a in zip(outs, acts))


# speedup vs baseline: 1.3990x; 1.3990x over previous
import jax
import jax.numpy as jnp
from jax.experimental import pallas as pl
from jax.experimental.pallas import tpu as pltpu

N_ROWS = 64
C = 256
_HW = (56 * 56, 28 * 28, 14 * 14, 7 * 7, 4 * 4)
_BLOCK = ((1, 64), (1, 256), (1, 256), (4, 256), (8, 256))


def _make_body(li, rpb, cpb, hw):
    def body(lids, chs, ain, aout):
        b0 = pl.program_id(0)
        b1 = pl.program_id(1)
        aout[...] = ain[...]
        for r in range(rpb):
            i = b0 * rpb + r
            lid = lids[i]
            c = chs[i]

            @pl.when((lid == li) & (c // cpb == b1))
            def _(r=r, c=c):
                aout[r, c % cpb, :] = jnp.full((hw,), -1000.0, jnp.float32)

    return body


def _ablate_level(li, a, layer_ids, ch):
    hw = _HW[li]
    rpb, cpb = _BLOCK[li]
    flat = a.reshape(N_ROWS, C, hw)
    spec = pl.BlockSpec((rpb, cpb, hw), lambda b0, b1, lids, chs: (b0, b1, 0))
    grid_spec = pltpu.PrefetchScalarGridSpec(
        num_scalar_prefetch=2,
        grid=(N_ROWS // rpb, C // cpb),
        in_specs=[spec],
        out_specs=spec,
    )
    out = pl.pallas_call(
        _make_body(li, rpb, cpb, hw),
        grid_spec=grid_spec,
        out_shape=jax.ShapeDtypeStruct(flat.shape, flat.dtype),
    )(layer_ids, ch, flat)
    return out.reshape(a.shape)


def kernel(act_0, act_1, act_2, act_3, act_pool, indices, x):
    del x
    acts = (act_0, act_1, act_2, act_3, act_pool)
    layer_ids = (indices // C).astype(jnp.int32)
    ch = (indices % C).astype(jnp.int32)
    return tuple(
        _ablate_level(li, a, layer_ids, ch) for li, a in enumerate(acts)
    )


# speedup vs baseline: 1.6815x; 1.2020x over previous
import jax
import jax.numpy as jnp
from jax.experimental import pallas as pl
from jax.experimental.pallas import tpu as pltpu

N_ROWS = 64
C = 256
_HW = (56 * 56, 28 * 28, 14 * 14, 7 * 7, 4 * 4)
_BLOCK = ((2, 256), (8, 256), (32, 256), (64, 256), (64, 256))


def _make_body(li, rpb, cpb, hw):
    def body(lids, chs, ain, aout):
        b0 = pl.program_id(0)
        b1 = pl.program_id(1)
        aout[...] = ain[...]
        for r in range(rpb):
            i = b0 * rpb + r
            lid = lids[i]
            c = chs[i]

            @pl.when((lid == li) & (c // cpb == b1))
            def _(r=r, c=c):
                aout[r, c % cpb, :] = jnp.full((hw,), -1000.0, jnp.float32)

    return body


def _ablate_level(li, a, layer_ids, ch):
    hw = _HW[li]
    rpb, cpb = _BLOCK[li]
    flat = a.reshape(N_ROWS, C, hw)
    spec = pl.BlockSpec((rpb, cpb, hw), lambda b0, b1, lids, chs: (b0, b1, 0))
    grid_spec = pltpu.PrefetchScalarGridSpec(
        num_scalar_prefetch=2,
        grid=(N_ROWS // rpb, C // cpb),
        in_specs=[spec],
        out_specs=spec,
    )
    out = pl.pallas_call(
        _make_body(li, rpb, cpb, hw),
        grid_spec=grid_spec,
        out_shape=jax.ShapeDtypeStruct(flat.shape, flat.dtype),
    )(layer_ids, ch, flat)
    return out.reshape(a.shape)


def kernel(act_0, act_1, act_2, act_3, act_pool, indices, x):
    del x
    acts = (act_0, act_1, act_2, act_3, act_pool)
    layer_ids = (indices // C).astype(jnp.int32)
    ch = (indices % C).astype(jnp.int32)
    return tuple(
        _ablate_level(li, a, layer_ids, ch) for li, a in enumerate(acts)
    )


# speedup vs baseline: 1.6839x; 1.0014x over previous
import jax
import jax.numpy as jnp
from jax.experimental import pallas as pl
from jax.experimental.pallas import tpu as pltpu

N_ROWS = 64
C = 256
_HW = (56 * 56, 28 * 28, 14 * 14, 7 * 7, 4 * 4)
_BLOCK = ((4, 256), (16, 256), (32, 256), (64, 256), (64, 256))


def _make_body(li, rpb, cpb, hw):
    def body(lids, chs, ain, aout):
        b0 = pl.program_id(0)
        b1 = pl.program_id(1)
        aout[...] = ain[...]
        for r in range(rpb):
            i = b0 * rpb + r
            lid = lids[i]
            c = chs[i]

            @pl.when((lid == li) & (c // cpb == b1))
            def _(r=r, c=c):
                aout[r, c % cpb, :] = jnp.full((hw,), -1000.0, jnp.float32)

    return body


def _ablate_level(li, a, layer_ids, ch):
    hw = _HW[li]
    rpb, cpb = _BLOCK[li]
    flat = a.reshape(N_ROWS, C, hw)
    spec = pl.BlockSpec((rpb, cpb, hw), lambda b0, b1, lids, chs: (b0, b1, 0))
    grid_spec = pltpu.PrefetchScalarGridSpec(
        num_scalar_prefetch=2,
        grid=(N_ROWS // rpb, C // cpb),
        in_specs=[spec],
        out_specs=spec,
    )
    out = pl.pallas_call(
        _make_body(li, rpb, cpb, hw),
        grid_spec=grid_spec,
        out_shape=jax.ShapeDtypeStruct(flat.shape, flat.dtype),
    )(layer_ids, ch, flat)
    return out.reshape(a.shape)


def kernel(act_0, act_1, act_2, act_3, act_pool, indices, x):
    del x
    acts = (act_0, act_1, act_2, act_3, act_pool)
    layer_ids = (indices // C).astype(jnp.int32)
    ch = (indices % C).astype(jnp.int32)
    return tuple(
        _ablate_level(li, a, layer_ids, ch) for li, a in enumerate(acts)
    )
